# baseline (speedup 1.0000x reference)
import functools

import jax
import jax.numpy as jnp
from jax import lax
from jax.experimental import pallas as pl
from jax.experimental.pallas import tpu as pltpu
from jax.experimental.pallas import tpu_sc as plsc

D = 768
E = 8
K = 2
FF = 4 * D
S = 2048

BLOCK = 128
G_MAX = (S * K) // BLOCK + (E - 1)
P_MAX = G_MAX * BLOCK
FFT = 768
NFF = FF // FFT

LANES = 128

NC, NS, L = 2, 16, 16
NW = NC * NS
CH = (S * K) // NW
T_CH = S // NW


def _gating_body(x_ref, wg_ref, bg_ref, ii_ref, ww_ref):
    s = jnp.dot(x_ref[...], wg_ref[...], preferred_element_type=jnp.float32)
    s = s + bg_ref[...]
    lane = lax.broadcasted_iota(jnp.int32, s.shape, 1)
    m1 = jnp.max(s, axis=1, keepdims=True)
    a1 = jnp.min(jnp.where(s == m1, lane, LANES), axis=1, keepdims=True)
    s2 = jnp.where(lane == a1, -jnp.inf, s)
    m2 = jnp.max(s2, axis=1, keepdims=True)
    a2 = jnp.min(jnp.where(s2 == m2, lane, LANES), axis=1, keepdims=True)
    w1 = 1.0 / (1.0 + jnp.exp(m2 - m1))
    w2 = 1.0 - w1
    ii_ref[...] = jnp.where(lane == 0, a1, jnp.where(lane == 1, a2, 0))
    ww_ref[...] = jnp.where(lane == 0, w1, jnp.where(lane == 1, w2, 0.0))


def _gating(x, wg_pad, bg_pad):
    return pl.pallas_call(
        _gating_body,
        out_shape=(
            jax.ShapeDtypeStruct((S, LANES), jnp.int32),
            jax.ShapeDtypeStruct((S, LANES), jnp.float32),
        ),
    )(x, wg_pad, bg_pad)


def _routing(e_flat):
    oh = (e_flat[:, None] == jnp.arange(E, dtype=jnp.int32)[None, :])
    oh = oh.astype(jnp.int32)
    rank = jnp.sum((jnp.cumsum(oh, axis=0) - oh) * oh, axis=1)
    counts = jnp.sum(oh, axis=0)
    padded = ((counts + BLOCK - 1) // BLOCK) * BLOCK
    row_bounds = jnp.cumsum(padded)
    base = row_bounds - padded
    p_flat = jnp.take(base, e_flat) + rank
    blk_bounds = row_bounds // BLOCK
    g = jnp.arange(G_MAX, dtype=jnp.int32)
    block_expert = jnp.sum(
        (g[:, None] >= blk_bounds[None, :]).astype(jnp.int32), axis=1)
    block_expert = jnp.minimum(block_expert, E - 1).astype(jnp.int32)
    return p_flat.astype(jnp.int32), block_expert


def _dispatch_body(x_hbm, t_hbm, p_hbm, xs_hbm, t_v, p_v, rows_v, sem):
    wid = lax.axis_index("s") * NC + lax.axis_index("c")
    base = wid * CH
    pltpu.sync_copy(t_hbm.at[pl.ds(base, CH)], t_v)
    pltpu.sync_copy(p_hbm.at[pl.ds(base, CH)], p_v)
    pltpu.async_copy(x_hbm.at[t_v], rows_v, sem).wait()
    pltpu.async_copy(rows_v, xs_hbm.at[p_v], sem).wait()


def _dispatch(x, t_flat, p_flat):
    mesh = plsc.VectorSubcoreMesh(core_axis_name="c", subcore_axis_name="s")
    return pl.kernel(
        _dispatch_body,
        mesh=mesh,
        out_type=jax.ShapeDtypeStruct((P_MAX, D), jnp.float32),
        scratch_types=[
            pltpu.VMEM((CH,), jnp.int32),
            pltpu.VMEM((CH,), jnp.int32),
            pltpu.VMEM((CH, D), jnp.float32),
            pltpu.SemaphoreType.DMA,
        ],
    )(x, t_flat, p_flat)


def _gemm_body(be_ref, xs_ref, w1_ref, b1_ref, w2_ref, b2_ref, ys_ref):
    h = jnp.dot(xs_ref[...], w1_ref[0], preferred_element_type=jnp.float32)
    h = h + b1_ref[0]
    h = 0.5 * h * (1.0 + lax.erf(h * 0.7071067811865476))
    y = jnp.dot(h, w2_ref[0], preferred_element_type=jnp.float32)
    ys_ref[...] = y + b2_ref[0]


def _grouped_gemm(block_expert, xs, W1, b1, W2, b2):
    grid_spec = pltpu.PrefetchScalarGridSpec(
        num_scalar_prefetch=1,
        grid=(G_MAX,),
        in_specs=[
            pl.BlockSpec((BLOCK, D), lambda g, be: (g, 0)),
            pl.BlockSpec((1, D, FF), lambda g, be: (be[g], 0, 0)),
            pl.BlockSpec((1, 1, FF), lambda g, be: (be[g], 0, 0)),
            pl.BlockSpec((1, FF, D), lambda g, be: (be[g], 0, 0)),
            pl.BlockSpec((1, 1, D), lambda g, be: (be[g], 0, 0)),
        ],
        out_specs=pl.BlockSpec((BLOCK, D), lambda g, be: (g, 0)),
    )
    return pl.pallas_call(
        _gemm_body,
        grid_spec=grid_spec,
        out_shape=jax.ShapeDtypeStruct((P_MAX, D), jnp.float32),
        compiler_params=pltpu.CompilerParams(
            dimension_semantics=("arbitrary",)),
    )(block_expert, xs, W1, b1.reshape(E, 1, FF),
      W2, b2.reshape(E, 1, D))


def _combine_body(ys_hbm, q0_hbm, q1_hbm, a_hbm, b_hbm,
                  q0_v, q1_v, a_v, b_v, sem):
    wid = lax.axis_index("s") * NC + lax.axis_index("c")
    base = wid * T_CH
    pltpu.sync_copy(q0_hbm.at[pl.ds(base, T_CH)], q0_v)
    pltpu.sync_copy(q1_hbm.at[pl.ds(base, T_CH)], q1_v)
    pltpu.async_copy(ys_hbm.at[q0_v], a_v, sem).wait()
    pltpu.async_copy(ys_hbm.at[q1_v], b_v, sem).wait()
    pltpu.sync_copy(a_v, a_hbm.at[pl.ds(base, T_CH)])
    pltpu.sync_copy(b_v, b_hbm.at[pl.ds(base, T_CH)])


def _combine_gather(ys, q0, q1):
    mesh = plsc.VectorSubcoreMesh(core_axis_name="c", subcore_axis_name="s")
    return pl.kernel(
        _combine_body,
        mesh=mesh,
        out_type=(
            jax.ShapeDtypeStruct((S, D), jnp.float32),
            jax.ShapeDtypeStruct((S, D), jnp.float32),
        ),
        scratch_types=[
            pltpu.VMEM((T_CH,), jnp.int32),
            pltpu.VMEM((T_CH,), jnp.int32),
            pltpu.VMEM((T_CH, D), jnp.float32),
            pltpu.VMEM((T_CH, D), jnp.float32),
            pltpu.SemaphoreType.DMA,
        ],
    )(ys, q0, q1)


def _wsum_body(a_ref, b_ref, w0_ref, w1_ref, out_ref):
    out_ref[...] = a_ref[...] * w0_ref[...] + b_ref[...] * w1_ref[...]


def _wsum(a, b, w0, w1):
    return pl.pallas_call(
        _wsum_body,
        out_shape=jax.ShapeDtypeStruct((S, D), jnp.float32),
    )(a, b, w0.reshape(S, 1), w1.reshape(S, 1))


def kernel(hidden_states, Wg, bg, W1, b1, W2, b2):
    x = hidden_states.reshape(S, D)

    wg_pad = jnp.zeros((D, LANES), jnp.float32).at[:, :E].set(Wg)
    bg_pad = jnp.full((1, LANES), -1e30, jnp.float32).at[0, :E].set(bg)
    ii, ww = _gating(x, wg_pad, bg_pad)

    e_pairs = ii[:, :K]
    w_pairs = ww[:, :K]
    e_flat = e_pairs.reshape(-1)
    p_flat, block_expert = _routing(e_flat)

    t_flat = jnp.repeat(jnp.arange(S, dtype=jnp.int32), K)
    xs = _dispatch(x, t_flat, p_flat)

    ys = _grouped_gemm(block_expert, xs, W1, b1, W2, b2)

    q = p_flat.reshape(S, K)
    a, b = _combine_gather(ys, q[:, 0], q[:, 1])
    out = _wsum(a, b, w_pairs[:, 0], w_pairs[:, 1])
    return out.reshape(hidden_states.shape)


# speedup vs baseline: 1.0504x; 1.0504x over previous
import functools

import jax
import jax.numpy as jnp
from jax import lax
from jax.experimental import pallas as pl
from jax.experimental.pallas import tpu as pltpu
from jax.experimental.pallas import tpu_sc as plsc

D = 768
E = 8
K = 2
FF = 4 * D
S = 2048

BLOCK = 256
G_MAX = (S * K) // BLOCK + (E - 1)
P_MAX = G_MAX * BLOCK
FFT = 768
NFF = FF // FFT

LANES = 128

NC, NS, L = 2, 16, 16
NW = NC * NS
CH = (S * K) // NW
T_CH = S // NW


def _gating_body(x_ref, wg_ref, bg_ref, ii_ref, ww_ref):
    s = jnp.dot(x_ref[...], wg_ref[...], preferred_element_type=jnp.float32)
    s = s + bg_ref[...]
    lane = lax.broadcasted_iota(jnp.int32, s.shape, 1)
    m1 = jnp.max(s, axis=1, keepdims=True)
    a1 = jnp.min(jnp.where(s == m1, lane, LANES), axis=1, keepdims=True)
    s2 = jnp.where(lane == a1, -jnp.inf, s)
    m2 = jnp.max(s2, axis=1, keepdims=True)
    a2 = jnp.min(jnp.where(s2 == m2, lane, LANES), axis=1, keepdims=True)
    w1 = 1.0 / (1.0 + jnp.exp(m2 - m1))
    w2 = 1.0 - w1
    ii_ref[...] = jnp.where(lane == 0, a1, jnp.where(lane == 1, a2, 0))
    ww_ref[...] = jnp.where(lane == 0, w1, jnp.where(lane == 1, w2, 0.0))


def _gating(x, wg_pad, bg_pad):
    return pl.pallas_call(
        _gating_body,
        out_shape=(
            jax.ShapeDtypeStruct((S, LANES), jnp.int32),
            jax.ShapeDtypeStruct((S, LANES), jnp.float32),
        ),
    )(x, wg_pad, bg_pad)


def _routing(e_flat):
    oh = (e_flat[:, None] == jnp.arange(E, dtype=jnp.int32)[None, :])
    oh = oh.astype(jnp.int32)
    rank = jnp.sum((jnp.cumsum(oh, axis=0) - oh) * oh, axis=1)
    counts = jnp.sum(oh, axis=0)
    padded = ((counts + BLOCK - 1) // BLOCK) * BLOCK
    row_bounds = jnp.cumsum(padded)
    base = row_bounds - padded
    p_flat = jnp.take(base, e_flat) + rank
    blk_bounds = row_bounds // BLOCK
    g = jnp.arange(G_MAX, dtype=jnp.int32)
    block_expert = jnp.sum(
        (g[:, None] >= blk_bounds[None, :]).astype(jnp.int32), axis=1)
    block_expert = jnp.minimum(block_expert, E - 1).astype(jnp.int32)
    return p_flat.astype(jnp.int32), block_expert


def _dispatch_body(x_hbm, t_hbm, p_hbm, wr_hbm, xs_hbm, ws_hbm,
                   t_v, p_v, rows_v, w_v, sem):
    wid = lax.axis_index("s") * NC + lax.axis_index("c")
    base = wid * CH
    pltpu.sync_copy(t_hbm.at[pl.ds(base, CH)], t_v)
    pltpu.sync_copy(p_hbm.at[pl.ds(base, CH)], p_v)
    pltpu.sync_copy(wr_hbm.at[pl.ds(base, CH)], w_v)
    pltpu.async_copy(x_hbm.at[t_v], rows_v, sem).wait()
    pltpu.async_copy(rows_v, xs_hbm.at[p_v], sem).wait()
    pltpu.async_copy(w_v, ws_hbm.at[p_v], sem).wait()


def _dispatch(x, t_flat, p_flat, w_rows):
    mesh = plsc.VectorSubcoreMesh(core_axis_name="c", subcore_axis_name="s")
    return pl.kernel(
        _dispatch_body,
        mesh=mesh,
        out_type=(
            jax.ShapeDtypeStruct((P_MAX, D), jnp.float32),
            jax.ShapeDtypeStruct((P_MAX, LANES), jnp.float32),
        ),
        scratch_types=[
            pltpu.VMEM((CH,), jnp.int32),
            pltpu.VMEM((CH,), jnp.int32),
            pltpu.VMEM((CH, D), jnp.float32),
            pltpu.VMEM((CH, LANES), jnp.float32),
            pltpu.SemaphoreType.DMA,
        ],
    )(x, t_flat, p_flat, w_rows)


def _gemm_body(be_ref, xs_ref, w1_ref, b1_ref, w2_ref, b2_ref, ws_ref,
               ys_ref):
    h = jnp.dot(xs_ref[...], w1_ref[0], preferred_element_type=jnp.float32)
    h = h + b1_ref[0]
    h = 0.5 * h * (1.0 + lax.erf(h * 0.7071067811865476))
    y = jnp.dot(h, w2_ref[0], preferred_element_type=jnp.float32)
    ys_ref[...] = (y + b2_ref[0]) * ws_ref[:, 0:1]


def _grouped_gemm(block_expert, xs, W1, b1, W2, b2, ws):
    grid_spec = pltpu.PrefetchScalarGridSpec(
        num_scalar_prefetch=1,
        grid=(G_MAX,),
        in_specs=[
            pl.BlockSpec((BLOCK, D), lambda g, be: (g, 0)),
            pl.BlockSpec((1, D, FF), lambda g, be: (be[g], 0, 0)),
            pl.BlockSpec((1, 1, FF), lambda g, be: (be[g], 0, 0)),
            pl.BlockSpec((1, FF, D), lambda g, be: (be[g], 0, 0)),
            pl.BlockSpec((1, 1, D), lambda g, be: (be[g], 0, 0)),
            pl.BlockSpec((BLOCK, LANES), lambda g, be: (g, 0)),
        ],
        out_specs=pl.BlockSpec((BLOCK, D), lambda g, be: (g, 0)),
    )
    return pl.pallas_call(
        _gemm_body,
        grid_spec=grid_spec,
        out_shape=jax.ShapeDtypeStruct((P_MAX, D), jnp.float32),
        compiler_params=pltpu.CompilerParams(
            dimension_semantics=("arbitrary",)),
    )(block_expert, xs, W1, b1.reshape(E, 1, FF),
      W2, b2.reshape(E, 1, D), ws)


def _combine_body(ys_hbm, q0_hbm, q1_hbm, out_hbm, q0_v, q1_v, a_v, b_v,
                  sem):
    wid = lax.axis_index("s") * NC + lax.axis_index("c")
    base = wid * T_CH
    pltpu.sync_copy(q0_hbm.at[pl.ds(base, T_CH)], q0_v)
    pltpu.sync_copy(q1_hbm.at[pl.ds(base, T_CH)], q1_v)
    pltpu.async_copy(ys_hbm.at[q0_v], a_v, sem).wait()
    pltpu.async_copy(ys_hbm.at[q1_v], b_v, sem).wait()

    def body(j, carry):
        for c in range(D // L):
            sl = pl.ds(c * L, L)
            a_v[j, sl] = a_v[j, sl] + b_v[j, sl]
        return carry

    lax.fori_loop(0, T_CH, body, 0)
    pltpu.sync_copy(a_v, out_hbm.at[pl.ds(base, T_CH)])


def _combine_gather(ys, q0, q1):
    mesh = plsc.VectorSubcoreMesh(core_axis_name="c", subcore_axis_name="s")
    return pl.kernel(
        _combine_body,
        mesh=mesh,
        out_type=jax.ShapeDtypeStruct((S, D), jnp.float32),
        scratch_types=[
            pltpu.VMEM((T_CH,), jnp.int32),
            pltpu.VMEM((T_CH,), jnp.int32),
            pltpu.VMEM((T_CH, D), jnp.float32),
            pltpu.VMEM((T_CH, D), jnp.float32),
            pltpu.SemaphoreType.DMA,
        ],
    )(ys, q0, q1)


def kernel(hidden_states, Wg, bg, W1, b1, W2, b2):
    x = hidden_states.reshape(S, D)

    wg_pad = jnp.zeros((D, LANES), jnp.float32).at[:, :E].set(Wg)
    bg_pad = jnp.full((1, LANES), -1e30, jnp.float32).at[0, :E].set(bg)
    ii, ww = _gating(x, wg_pad, bg_pad)

    e_pairs = ii[:, :K]
    w_pairs = ww[:, :K]
    e_flat = e_pairs.reshape(-1)
    p_flat, block_expert = _routing(e_flat)

    t_flat = jnp.repeat(jnp.arange(S, dtype=jnp.int32), K)
    w_rows = jnp.broadcast_to(w_pairs.reshape(-1)[:, None], (S * K, LANES))
    xs, ws = _dispatch(x, t_flat, p_flat, w_rows)

    ys = _grouped_gemm(block_expert, xs, W1, b1, W2, b2, ws)

    q = p_flat.reshape(S, K)
    out = _combine_gather(ys, q[:, 0], q[:, 1])
    return out.reshape(hidden_states.shape)


# speedup vs baseline: 1.0510x; 1.0005x over previous
import functools

import jax
import jax.numpy as jnp
from jax import lax
from jax.experimental import pallas as pl
from jax.experimental.pallas import tpu as pltpu
from jax.experimental.pallas import tpu_sc as plsc

D = 768
E = 8
K = 2
FF = 4 * D
S = 2048

BLOCK = 256
G_MAX = (S * K) // BLOCK + (E - 1)
P_MAX = G_MAX * BLOCK
FFT = 768
NFF = FF // FFT

LANES = 128

NC, NS, L = 2, 16, 16
NW = NC * NS
CH = (S * K) // NW
T_CH = S // NW


def _gating_body(x_ref, wg_ref, bg_ref, ii_ref, ww_ref):
    s = jnp.dot(x_ref[...], wg_ref[...], preferred_element_type=jnp.float32)
    s = s + bg_ref[...]
    lane = lax.broadcasted_iota(jnp.int32, s.shape, 1)
    m1 = jnp.max(s, axis=1, keepdims=True)
    a1 = jnp.min(jnp.where(s == m1, lane, LANES), axis=1, keepdims=True)
    s2 = jnp.where(lane == a1, -jnp.inf, s)
    m2 = jnp.max(s2, axis=1, keepdims=True)
    a2 = jnp.min(jnp.where(s2 == m2, lane, LANES), axis=1, keepdims=True)
    w1 = 1.0 / (1.0 + jnp.exp(m2 - m1))
    w2 = 1.0 - w1
    ii_ref[...] = jnp.where(lane == 0, a1, jnp.where(lane == 1, a2, 0))
    ww_ref[...] = jnp.where(lane == 0, w1, jnp.where(lane == 1, w2, 0.0))


def _gating(x, wg_pad, bg_pad):
    return pl.pallas_call(
        _gating_body,
        out_shape=(
            jax.ShapeDtypeStruct((S, LANES), jnp.int32),
            jax.ShapeDtypeStruct((S, LANES), jnp.float32),
        ),
    )(x, wg_pad, bg_pad)


def _routing(e_flat):
    oh = (jnp.arange(E, dtype=jnp.int32)[:, None] == e_flat[None, :])
    oh = oh.astype(jnp.int32)
    rank = jnp.sum((jnp.cumsum(oh, axis=1) - oh) * oh, axis=0)
    counts = jnp.sum(oh, axis=1)
    padded = ((counts + BLOCK - 1) // BLOCK) * BLOCK
    row_bounds = jnp.cumsum(padded)
    base = row_bounds - padded
    p_flat = jnp.take(base, e_flat) + rank
    blk_bounds = row_bounds // BLOCK
    g = jnp.arange(G_MAX, dtype=jnp.int32)
    block_expert = jnp.sum(
        (g[:, None] >= blk_bounds[None, :]).astype(jnp.int32), axis=1)
    block_expert = jnp.minimum(block_expert, E - 1).astype(jnp.int32)
    return p_flat.astype(jnp.int32), block_expert


def _dispatch_body(x_hbm, t_hbm, p_hbm, wr_hbm, xs_hbm, ws_hbm,
                   t_v, p_v, rows_v, w_v, sem):
    wid = lax.axis_index("s") * NC + lax.axis_index("c")
    base = wid * CH
    pltpu.sync_copy(t_hbm.at[pl.ds(base, CH)], t_v)
    pltpu.sync_copy(p_hbm.at[pl.ds(base, CH)], p_v)
    pltpu.sync_copy(wr_hbm.at[pl.ds(base, CH)], w_v)
    pltpu.async_copy(x_hbm.at[t_v], rows_v, sem).wait()
    pltpu.async_copy(rows_v, xs_hbm.at[p_v], sem).wait()
    pltpu.async_copy(w_v, ws_hbm.at[p_v], sem).wait()


def _dispatch(x, t_flat, p_flat, w_rows):
    mesh = plsc.VectorSubcoreMesh(core_axis_name="c", subcore_axis_name="s")
    return pl.kernel(
        _dispatch_body,
        mesh=mesh,
        out_type=(
            jax.ShapeDtypeStruct((P_MAX, D), jnp.float32),
            jax.ShapeDtypeStruct((P_MAX, LANES), jnp.float32),
        ),
        scratch_types=[
            pltpu.VMEM((CH,), jnp.int32),
            pltpu.VMEM((CH,), jnp.int32),
            pltpu.VMEM((CH, D), jnp.float32),
            pltpu.VMEM((CH, LANES), jnp.float32),
            pltpu.SemaphoreType.DMA,
        ],
    )(x, t_flat, p_flat, w_rows)


def _gemm_body(be_ref, xs_ref, w1_ref, b1_ref, w2_ref, b2_ref, ws_ref,
               ys_ref):
    h = jnp.dot(xs_ref[...], w1_ref[0], preferred_element_type=jnp.float32)
    h = h + b1_ref[0]
    h = 0.5 * h * (1.0 + lax.erf(h * 0.7071067811865476))
    y = jnp.dot(h, w2_ref[0], preferred_element_type=jnp.float32)
    ys_ref[...] = (y + b2_ref[0]) * ws_ref[:, 0:1]


def _grouped_gemm(block_expert, xs, W1, b1, W2, b2, ws):
    grid_spec = pltpu.PrefetchScalarGridSpec(
        num_scalar_prefetch=1,
        grid=(G_MAX,),
        in_specs=[
            pl.BlockSpec((BLOCK, D), lambda g, be: (g, 0)),
            pl.BlockSpec((1, D, FF), lambda g, be: (be[g], 0, 0)),
            pl.BlockSpec((1, 1, FF), lambda g, be: (be[g], 0, 0)),
            pl.BlockSpec((1, FF, D), lambda g, be: (be[g], 0, 0)),
            pl.BlockSpec((1, 1, D), lambda g, be: (be[g], 0, 0)),
            pl.BlockSpec((BLOCK, LANES), lambda g, be: (g, 0)),
        ],
        out_specs=pl.BlockSpec((BLOCK, D), lambda g, be: (g, 0)),
    )
    return pl.pallas_call(
        _gemm_body,
        grid_spec=grid_spec,
        out_shape=jax.ShapeDtypeStruct((P_MAX, D), jnp.float32),
        compiler_params=pltpu.CompilerParams(
            dimension_semantics=("arbitrary",)),
    )(block_expert, xs, W1, b1.reshape(E, 1, FF),
      W2, b2.reshape(E, 1, D), ws)


def _combine_body(ys_hbm, q0_hbm, q1_hbm, out_hbm, q0_v, q1_v, a_v, b_v,
                  sem):
    wid = lax.axis_index("s") * NC + lax.axis_index("c")
    base = wid * T_CH
    pltpu.sync_copy(q0_hbm.at[pl.ds(base, T_CH)], q0_v)
    pltpu.sync_copy(q1_hbm.at[pl.ds(base, T_CH)], q1_v)
    pltpu.async_copy(ys_hbm.at[q0_v], a_v, sem).wait()
    pltpu.async_copy(ys_hbm.at[q1_v], b_v, sem).wait()

    def body(j, carry):
        for c in range(D // L):
            sl = pl.ds(c * L, L)
            a_v[j, sl] = a_v[j, sl] + b_v[j, sl]
        return carry

    lax.fori_loop(0, T_CH, body, 0)
    pltpu.sync_copy(a_v, out_hbm.at[pl.ds(base, T_CH)])


def _combine_gather(ys, q0, q1):
    mesh = plsc.VectorSubcoreMesh(core_axis_name="c", subcore_axis_name="s")
    return pl.kernel(
        _combine_body,
        mesh=mesh,
        out_type=jax.ShapeDtypeStruct((S, D), jnp.float32),
        scratch_types=[
            pltpu.VMEM((T_CH,), jnp.int32),
            pltpu.VMEM((T_CH,), jnp.int32),
            pltpu.VMEM((T_CH, D), jnp.float32),
            pltpu.VMEM((T_CH, D), jnp.float32),
            pltpu.SemaphoreType.DMA,
        ],
    )(ys, q0, q1)


def kernel(hidden_states, Wg, bg, W1, b1, W2, b2):
    x = hidden_states.reshape(S, D)

    wg_pad = jnp.zeros((D, LANES), jnp.float32).at[:, :E].set(Wg)
    bg_pad = jnp.full((1, LANES), -1e30, jnp.float32).at[0, :E].set(bg)
    ii, ww = _gating(x, wg_pad, bg_pad)

    e_pairs = ii[:, :K]
    w_pairs = ww[:, :K]
    e_flat = e_pairs.reshape(-1)
    p_flat, block_expert = _routing(e_flat)

    t_flat = jnp.repeat(jnp.arange(S, dtype=jnp.int32), K)
    w_rows = jnp.broadcast_to(w_pairs.reshape(-1)[:, None], (S * K, LANES))
    xs, ws = _dispatch(x, t_flat, p_flat, w_rows)

    ys = _grouped_gemm(block_expert, xs, W1, b1, W2, b2, ws)

    q = p_flat.reshape(S, K)
    out = _combine_gather(ys, q[:, 0], q[:, 1])
    return out.reshape(hidden_states.shape)


# speedup vs baseline: 1.2281x; 1.1685x over previous
import functools

import jax
import jax.numpy as jnp
from jax import lax
from jax.experimental import pallas as pl
from jax.experimental.pallas import tpu as pltpu
from jax.experimental.pallas import tpu_sc as plsc

D = 768
E = 8
K = 2
FF = 4 * D
S = 2048

BLOCK = 256
G_MAX = (S * K) // BLOCK + (E - 1)
P_MAX = G_MAX * BLOCK
FFT = 768
NFF = FF // FFT

LANES = 128

NC, NS, L = 2, 16, 16
NW = NC * NS
CH = (S * K) // NW
T_CH = S // NW


def _gating_body(x_ref, wg_ref, bg_ref, ii_ref, ww_ref):
    s = jnp.dot(x_ref[...], wg_ref[...], preferred_element_type=jnp.float32)
    s = s + bg_ref[...]
    lane = lax.broadcasted_iota(jnp.int32, s.shape, 1)
    m1 = jnp.max(s, axis=1, keepdims=True)
    a1 = jnp.min(jnp.where(s == m1, lane, LANES), axis=1, keepdims=True)
    s2 = jnp.where(lane == a1, -jnp.inf, s)
    m2 = jnp.max(s2, axis=1, keepdims=True)
    a2 = jnp.min(jnp.where(s2 == m2, lane, LANES), axis=1, keepdims=True)
    w1 = 1.0 / (1.0 + jnp.exp(m2 - m1))
    w2 = 1.0 - w1
    ii_ref[...] = jnp.where(lane == 0, a1, jnp.where(lane == 1, a2, 0))
    ww_ref[...] = jnp.where(lane == 0, w1, jnp.where(lane == 1, w2, 0.0))


def _gating(x, wg_pad, bg_pad):
    return pl.pallas_call(
        _gating_body,
        out_shape=(
            jax.ShapeDtypeStruct((S, LANES), jnp.int32),
            jax.ShapeDtypeStruct((S, LANES), jnp.float32),
        ),
    )(x, wg_pad, bg_pad)


def _routing(e_flat):
    oh = (jnp.arange(E, dtype=jnp.int32)[:, None] == e_flat[None, :])
    oh = oh.astype(jnp.int32)
    rank = jnp.sum((jnp.cumsum(oh, axis=1) - oh) * oh, axis=0)
    counts = jnp.sum(oh, axis=1)
    padded = ((counts + BLOCK - 1) // BLOCK) * BLOCK
    row_bounds = jnp.cumsum(padded)
    base = row_bounds - padded
    p_flat = jnp.take(base, e_flat) + rank
    blk_bounds = row_bounds // BLOCK
    g = jnp.arange(G_MAX, dtype=jnp.int32)
    block_expert = jnp.sum(
        (g[:, None] >= blk_bounds[None, :]).astype(jnp.int32), axis=1)
    last_e = jnp.max(jnp.where(counts > 0, jnp.arange(E, dtype=jnp.int32), 0))
    block_expert = jnp.minimum(block_expert, last_e).astype(jnp.int32)
    first = jnp.concatenate([
        jnp.ones((1,), jnp.int32),
        (block_expert[1:] != block_expert[:-1]).astype(jnp.int32)])
    seg = jnp.cumsum(first) - 1
    seg_expert = jnp.zeros((G_MAX,), jnp.int32).at[seg].set(block_expert)
    nblocks = jnp.full((G_MAX,), blk_bounds[E - 1], jnp.int32)
    meta = jnp.stack([block_expert, first, seg, seg_expert, nblocks])
    return p_flat.astype(jnp.int32), meta


def _dispatch_body(x_hbm, t_hbm, p_hbm, wr_hbm, xs_hbm, ws_hbm,
                   t_v, p_v, rows_v, w_v, sem):
    wid = lax.axis_index("s") * NC + lax.axis_index("c")
    base = wid * CH
    pltpu.sync_copy(t_hbm.at[pl.ds(base, CH)], t_v)
    pltpu.sync_copy(p_hbm.at[pl.ds(base, CH)], p_v)
    pltpu.sync_copy(wr_hbm.at[pl.ds(base, CH)], w_v)
    pltpu.async_copy(x_hbm.at[t_v], rows_v, sem).wait()
    pltpu.async_copy(rows_v, xs_hbm.at[p_v], sem).wait()
    pltpu.async_copy(w_v, ws_hbm.at[p_v], sem).wait()


def _dispatch(x, t_flat, p_flat, w_rows):
    mesh = plsc.VectorSubcoreMesh(core_axis_name="c", subcore_axis_name="s")
    return pl.kernel(
        _dispatch_body,
        mesh=mesh,
        out_type=(
            jax.ShapeDtypeStruct((P_MAX, D), jnp.float32),
            jax.ShapeDtypeStruct((P_MAX, LANES), jnp.float32),
        ),
        scratch_types=[
            pltpu.VMEM((CH,), jnp.int32),
            pltpu.VMEM((CH,), jnp.int32),
            pltpu.VMEM((CH, D), jnp.float32),
            pltpu.VMEM((CH, LANES), jnp.float32),
            pltpu.SemaphoreType.DMA,
        ],
    )(x, t_flat, p_flat, w_rows)


def _gemm_body(meta_ref, xs_ref, w1_hbm, b1_ref, w2_hbm, b2_ref, ws_ref,
               ys_ref, w1_buf, w2_buf, wsem):
    g = pl.program_id(0)
    s = meta_ref[2, g]
    first = meta_ref[1, g]
    nseg = meta_ref[2, G_MAX - 1] + 1
    nblocks = meta_ref[4, 0]
    buf = lax.rem(s, 2)

    def w_copies(e_, slot):
        return (pltpu.make_async_copy(w1_hbm.at[e_], w1_buf.at[slot],
                                      wsem.at[slot]),
                pltpu.make_async_copy(w2_hbm.at[e_], w2_buf.at[slot],
                                      wsem.at[slot]))

    @pl.when(first == 1)
    def _():
        @pl.when(s == 0)
        def _():
            c1, c2 = w_copies(meta_ref[3, 0], 0)
            c1.start()
            c2.start()

        c1, c2 = w_copies(meta_ref[3, s], buf)
        c1.wait()
        c2.wait()

        @pl.when(s + 1 < nseg)
        def _():
            n1, n2 = w_copies(meta_ref[3, s + 1], 1 - buf)
            n1.start()
            n2.start()

    @pl.when(g < nblocks)
    def _():
        h = jnp.dot(xs_ref[...], w1_buf[buf],
                    preferred_element_type=jnp.float32)
        h = h + b1_ref[0]
        h = 0.5 * h * (1.0 + lax.erf(h * 0.7071067811865476))
        y = jnp.dot(h, w2_buf[buf], preferred_element_type=jnp.float32)
        ys_ref[...] = (y + b2_ref[0]) * ws_ref[:, 0:1]


def _grouped_gemm(meta, xs, W1, b1, W2, b2, ws):
    grid_spec = pltpu.PrefetchScalarGridSpec(
        num_scalar_prefetch=1,
        grid=(G_MAX,),
        in_specs=[
            pl.BlockSpec((BLOCK, D), lambda g, m: (g, 0)),
            pl.BlockSpec(memory_space=pl.ANY),
            pl.BlockSpec((1, 1, FF), lambda g, m: (m[0, g], 0, 0)),
            pl.BlockSpec(memory_space=pl.ANY),
            pl.BlockSpec((1, 1, D), lambda g, m: (m[0, g], 0, 0)),
            pl.BlockSpec((BLOCK, LANES), lambda g, m: (g, 0)),
        ],
        out_specs=pl.BlockSpec((BLOCK, D), lambda g, m: (g, 0)),
        scratch_shapes=[
            pltpu.VMEM((2, D, FF), jnp.float32),
            pltpu.VMEM((2, FF, D), jnp.float32),
            pltpu.SemaphoreType.DMA((2,)),
        ],
    )
    return pl.pallas_call(
        _gemm_body,
        grid_spec=grid_spec,
        out_shape=jax.ShapeDtypeStruct((P_MAX, D), jnp.float32),
        compiler_params=pltpu.CompilerParams(
            dimension_semantics=("arbitrary",)),
    )(meta, xs, W1, b1.reshape(E, 1, FF),
      W2, b2.reshape(E, 1, D), ws)


def _combine_body(ys_hbm, q0_hbm, q1_hbm, out_hbm, q0_v, q1_v, a_v, b_v,
                  sem):
    wid = lax.axis_index("s") * NC + lax.axis_index("c")
    base = wid * T_CH
    pltpu.sync_copy(q0_hbm.at[pl.ds(base, T_CH)], q0_v)
    pltpu.sync_copy(q1_hbm.at[pl.ds(base, T_CH)], q1_v)
    pltpu.async_copy(ys_hbm.at[q0_v], a_v, sem).wait()
    pltpu.async_copy(ys_hbm.at[q1_v], b_v, sem).wait()

    def body(j, carry):
        for c in range(D // L):
            sl = pl.ds(c * L, L)
            a_v[j, sl] = a_v[j, sl] + b_v[j, sl]
        return carry

    lax.fori_loop(0, T_CH, body, 0)
    pltpu.sync_copy(a_v, out_hbm.at[pl.ds(base, T_CH)])


def _combine_gather(ys, q0, q1):
    mesh = plsc.VectorSubcoreMesh(core_axis_name="c", subcore_axis_name="s")
    return pl.kernel(
        _combine_body,
        mesh=mesh,
        out_type=jax.ShapeDtypeStruct((S, D), jnp.float32),
        scratch_types=[
            pltpu.VMEM((T_CH,), jnp.int32),
            pltpu.VMEM((T_CH,), jnp.int32),
            pltpu.VMEM((T_CH, D), jnp.float32),
            pltpu.VMEM((T_CH, D), jnp.float32),
            pltpu.SemaphoreType.DMA,
        ],
    )(ys, q0, q1)


def kernel(hidden_states, Wg, bg, W1, b1, W2, b2):
    x = hidden_states.reshape(S, D)

    wg_pad = jnp.zeros((D, LANES), jnp.float32).at[:, :E].set(Wg)
    bg_pad = jnp.full((1, LANES), -1e30, jnp.float32).at[0, :E].set(bg)
    ii, ww = _gating(x, wg_pad, bg_pad)

    e_pairs = ii[:, :K]
    w_pairs = ww[:, :K]
    e_flat = e_pairs.reshape(-1)
    p_flat, meta = _routing(e_flat)

    t_flat = jnp.repeat(jnp.arange(S, dtype=jnp.int32), K)
    w_rows = jnp.broadcast_to(w_pairs.reshape(-1)[:, None], (S * K, LANES))
    xs, ws = _dispatch(x, t_flat, p_flat, w_rows)

    ys = _grouped_gemm(meta, xs, W1, b1, W2, b2, ws)

    q = p_flat.reshape(S, K)
    out = _combine_gather(ys, q[:, 0], q[:, 1])
    return out.reshape(hidden_states.shape)


# speedup vs baseline: 1.2465x; 1.0150x over previous
import functools

import jax
import jax.numpy as jnp
from jax import lax
from jax.experimental import pallas as pl
from jax.experimental.pallas import tpu as pltpu
from jax.experimental.pallas import tpu_sc as plsc

D = 768
E = 8
K = 2
FF = 4 * D
S = 2048

BLOCK = 256
G_MAX = (S * K) // BLOCK + (E - 1)
P_MAX = G_MAX * BLOCK
FFT = 768
NFF = FF // FFT

LANES = 128

NC, NS, L = 2, 16, 16
NW = NC * NS
CH = (S * K) // NW
T_CH = S // NW


def _gating_body(x_ref, wg_ref, bg_ref, ii_ref, ww_ref):
    s = jnp.dot(x_ref[...], wg_ref[...], preferred_element_type=jnp.float32)
    s = s + bg_ref[...]
    lane = lax.broadcasted_iota(jnp.int32, s.shape, 1)
    m1 = jnp.max(s, axis=1, keepdims=True)
    a1 = jnp.min(jnp.where(s == m1, lane, LANES), axis=1, keepdims=True)
    s2 = jnp.where(lane == a1, -jnp.inf, s)
    m2 = jnp.max(s2, axis=1, keepdims=True)
    a2 = jnp.min(jnp.where(s2 == m2, lane, LANES), axis=1, keepdims=True)
    w1 = 1.0 / (1.0 + jnp.exp(m2 - m1))
    w2 = 1.0 - w1
    ii_ref[...] = jnp.where(lane == 0, a1, jnp.where(lane == 1, a2, 0))
    ww_ref[...] = jnp.concatenate(
        [jnp.broadcast_to(w1[:, None, :], (w1.shape[0], 1, LANES)),
         jnp.broadcast_to(w2[:, None, :], (w2.shape[0], 1, LANES))], axis=1)


def _gating(x, wg_pad, bg_pad):
    return pl.pallas_call(
        _gating_body,
        out_shape=(
            jax.ShapeDtypeStruct((S, LANES), jnp.int32),
            jax.ShapeDtypeStruct((S, K, LANES), jnp.float32),
        ),
    )(x, wg_pad, bg_pad)


def _routing(e_flat):
    oh = (jnp.arange(E, dtype=jnp.int32)[:, None] == e_flat[None, :])
    oh = oh.astype(jnp.int32)
    rank = jnp.sum((jnp.cumsum(oh, axis=1) - oh) * oh, axis=0)
    counts = jnp.sum(oh, axis=1)
    padded = ((counts + BLOCK - 1) // BLOCK) * BLOCK
    row_bounds = jnp.cumsum(padded)
    base = row_bounds - padded
    p_flat = jnp.take(base, e_flat) + rank
    blk_bounds = row_bounds // BLOCK
    g = jnp.arange(G_MAX, dtype=jnp.int32)
    block_expert = jnp.sum(
        (g[:, None] >= blk_bounds[None, :]).astype(jnp.int32), axis=1)
    last_e = jnp.max(jnp.where(counts > 0, jnp.arange(E, dtype=jnp.int32), 0))
    block_expert = jnp.minimum(block_expert, last_e).astype(jnp.int32)
    first = jnp.concatenate([
        jnp.ones((1,), jnp.int32),
        (block_expert[1:] != block_expert[:-1]).astype(jnp.int32)])
    seg = jnp.cumsum(first) - 1
    seg_expert = jnp.zeros((G_MAX,), jnp.int32).at[seg].set(block_expert)
    nblocks = jnp.full((G_MAX,), blk_bounds[E - 1], jnp.int32)
    meta = jnp.stack([block_expert, first, seg, seg_expert, nblocks])
    return p_flat.astype(jnp.int32), meta


def _dispatch_body(x_hbm, t_hbm, p_hbm, wr_hbm, xs_hbm, ws_hbm,
                   t_v, p_v, rows_v, w_v, sem, sem2):
    wid = lax.axis_index("s") * NC + lax.axis_index("c")
    base = wid * CH
    pltpu.sync_copy(t_hbm.at[pl.ds(base, CH)], t_v)
    pltpu.sync_copy(p_hbm.at[pl.ds(base, CH)], p_v)
    pltpu.sync_copy(wr_hbm.at[pl.ds(base, CH)], w_v)
    gat = pltpu.async_copy(x_hbm.at[t_v], rows_v, sem)
    wsc = pltpu.async_copy(w_v, ws_hbm.at[p_v], sem2)
    gat.wait()
    pltpu.async_copy(rows_v, xs_hbm.at[p_v], sem).wait()
    wsc.wait()


def _dispatch(x, t_flat, p_flat, w_rows):
    mesh = plsc.VectorSubcoreMesh(core_axis_name="c", subcore_axis_name="s")
    return pl.kernel(
        _dispatch_body,
        mesh=mesh,
        out_type=(
            jax.ShapeDtypeStruct((P_MAX, D), jnp.float32),
            jax.ShapeDtypeStruct((P_MAX, LANES), jnp.float32),
        ),
        scratch_types=[
            pltpu.VMEM((CH,), jnp.int32),
            pltpu.VMEM((CH,), jnp.int32),
            pltpu.VMEM((CH, D), jnp.float32),
            pltpu.VMEM((CH, LANES), jnp.float32),
            pltpu.SemaphoreType.DMA,
            pltpu.SemaphoreType.DMA,
        ],
    )(x, t_flat, p_flat, w_rows)


def _gemm_body(meta_ref, xs_ref, w1_hbm, b1_ref, w2_hbm, b2_ref, ws_ref,
               ys_ref, w1_buf, w2_buf, wsem):
    g = pl.program_id(0)
    s = meta_ref[2, g]
    first = meta_ref[1, g]
    nseg = meta_ref[2, G_MAX - 1] + 1
    nblocks = meta_ref[4, 0]
    buf = lax.rem(s, 2)

    def w_copies(e_, slot):
        return (pltpu.make_async_copy(w1_hbm.at[e_], w1_buf.at[slot],
                                      wsem.at[slot]),
                pltpu.make_async_copy(w2_hbm.at[e_], w2_buf.at[slot],
                                      wsem.at[slot]))

    @pl.when(first == 1)
    def _():
        @pl.when(s == 0)
        def _():
            c1, c2 = w_copies(meta_ref[3, 0], 0)
            c1.start()
            c2.start()

        c1, c2 = w_copies(meta_ref[3, s], buf)
        c1.wait()
        c2.wait()

        @pl.when(s + 1 < nseg)
        def _():
            n1, n2 = w_copies(meta_ref[3, s + 1], 1 - buf)
            n1.start()
            n2.start()

    @pl.when(g < nblocks)
    def _():
        h = jnp.dot(xs_ref[...], w1_buf[buf],
                    preferred_element_type=jnp.float32)
        h = h + b1_ref[0]
        h = 0.5 * h * (1.0 + lax.erf(h * 0.7071067811865476))
        y = jnp.dot(h, w2_buf[buf], preferred_element_type=jnp.float32)
        ys_ref[...] = (y + b2_ref[0]) * ws_ref[:, 0:1]


def _grouped_gemm(meta, xs, W1, b1, W2, b2, ws):
    grid_spec = pltpu.PrefetchScalarGridSpec(
        num_scalar_prefetch=1,
        grid=(G_MAX,),
        in_specs=[
            pl.BlockSpec((BLOCK, D), lambda g, m: (g, 0)),
            pl.BlockSpec(memory_space=pl.ANY),
            pl.BlockSpec((1, 1, FF), lambda g, m: (m[0, g], 0, 0)),
            pl.BlockSpec(memory_space=pl.ANY),
            pl.BlockSpec((1, 1, D), lambda g, m: (m[0, g], 0, 0)),
            pl.BlockSpec((BLOCK, LANES), lambda g, m: (g, 0)),
        ],
        out_specs=pl.BlockSpec((BLOCK, D), lambda g, m: (g, 0)),
        scratch_shapes=[
            pltpu.VMEM((2, D, FF), jnp.float32),
            pltpu.VMEM((2, FF, D), jnp.float32),
            pltpu.SemaphoreType.DMA((2,)),
        ],
    )
    return pl.pallas_call(
        _gemm_body,
        grid_spec=grid_spec,
        out_shape=jax.ShapeDtypeStruct((P_MAX, D), jnp.float32),
        compiler_params=pltpu.CompilerParams(
            dimension_semantics=("arbitrary",)),
    )(meta, xs, W1, b1.reshape(E, 1, FF),
      W2, b2.reshape(E, 1, D), ws)


def _combine_body(ys_hbm, q0_hbm, q1_hbm, out_hbm, q0_v, q1_v, a_v, b_v,
                  sem):
    wid = lax.axis_index("s") * NC + lax.axis_index("c")
    base = wid * T_CH
    pltpu.sync_copy(q0_hbm.at[pl.ds(base, T_CH)], q0_v)
    pltpu.sync_copy(q1_hbm.at[pl.ds(base, T_CH)], q1_v)
    pltpu.async_copy(ys_hbm.at[q0_v], a_v, sem).wait()
    pltpu.async_copy(ys_hbm.at[q1_v], b_v, sem).wait()

    def body(j, carry):
        for c in range(D // L):
            sl = pl.ds(c * L, L)
            a_v[j, sl] = a_v[j, sl] + b_v[j, sl]
        return carry

    lax.fori_loop(0, T_CH, body, 0)
    pltpu.sync_copy(a_v, out_hbm.at[pl.ds(base, T_CH)])


def _combine_gather(ys, q0, q1):
    mesh = plsc.VectorSubcoreMesh(core_axis_name="c", subcore_axis_name="s")
    return pl.kernel(
        _combine_body,
        mesh=mesh,
        out_type=jax.ShapeDtypeStruct((S, D), jnp.float32),
        scratch_types=[
            pltpu.VMEM((T_CH,), jnp.int32),
            pltpu.VMEM((T_CH,), jnp.int32),
            pltpu.VMEM((T_CH, D), jnp.float32),
            pltpu.VMEM((T_CH, D), jnp.float32),
            pltpu.SemaphoreType.DMA,
        ],
    )(ys, q0, q1)


def kernel(hidden_states, Wg, bg, W1, b1, W2, b2):
    x = hidden_states.reshape(S, D)

    wg_pad = jnp.zeros((D, LANES), jnp.float32).at[:, :E].set(Wg)
    bg_pad = jnp.full((1, LANES), -1e30, jnp.float32).at[0, :E].set(bg)
    ii, ww = _gating(x, wg_pad, bg_pad)

    e_pairs = ii[:, :K]
    e_flat = e_pairs.reshape(-1)
    p_flat, meta = _routing(e_flat)

    t_flat = jnp.repeat(jnp.arange(S, dtype=jnp.int32), K)
    w_rows = ww.reshape(S * K, LANES)
    xs, ws = _dispatch(x, t_flat, p_flat, w_rows)

    ys = _grouped_gemm(meta, xs, W1, b1, W2, b2, ws)

    q = p_flat.reshape(S, K)
    out = _combine_gather(ys, q[:, 0], q[:, 1])
    return out.reshape(hidden_states.shape)
